# baseline (speedup 1.0000x reference)
import functools

import jax
import jax.numpy as jnp
from jax import lax
from jax.experimental import pallas as pl
from jax.experimental.pallas import tpu as pltpu
from jax.experimental.pallas import tpu_sc as plsc

VOCAB = 1000000
EMBED_DIM = 64
BATCH = 4096
P_LEN = 50

NUM_IDX = BATCH * P_LEN
NUM_WORKERS = 32
PER_WORKER = NUM_IDX // NUM_WORKERS
CHUNK = 128
NCHUNK = PER_WORKER // CHUNK


def _build_gather():
    mesh = plsc.VectorSubcoreMesh(core_axis_name="c", subcore_axis_name="s")

    @functools.partial(
        pl.kernel,
        mesh=mesh,
        compiler_params=pltpu.CompilerParams(use_tc_tiling_on_sc=False),
        out_type=jax.ShapeDtypeStruct((NUM_IDX, EMBED_DIM), jnp.float32),
        scratch_types=[
            pltpu.VMEM((NCHUNK, CHUNK), jnp.int32),
            pltpu.VMEM((CHUNK, EMBED_DIM), jnp.float32),
            pltpu.SemaphoreType.DMA,
        ],
    )
    def gather_kernel(table_hbm, idx_hbm, out_hbm, idx_v, rows_v, sem):
        wid = lax.axis_index("s") * 2 + lax.axis_index("c")
        base = wid * PER_WORKER
        pltpu.sync_copy(idx_hbm.at[wid], idx_v)

        def step(j, carry):
            pltpu.async_copy(table_hbm.at[idx_v.at[j]], rows_v, sem).wait()
            pltpu.sync_copy(rows_v, out_hbm.at[pl.ds(base + j * CHUNK, CHUNK)])
            return carry

        lax.fori_loop(0, NCHUNK, step, 0)

    return gather_kernel


_gather = _build_gather()


def kernel(x, pw_idxs, qw_idxs, p_mask, q_mask, word_vectors):
    idx = x.astype(jnp.int32).reshape(NUM_WORKERS, NCHUNK, CHUNK)
    out = _gather(word_vectors, idx)
    return out.reshape(BATCH, P_LEN, EMBED_DIM)


# speedup vs baseline: 1.0453x; 1.0453x over previous
import functools

import jax
import jax.numpy as jnp
from jax import lax
from jax.experimental import pallas as pl
from jax.experimental.pallas import tpu as pltpu
from jax.experimental.pallas import tpu_sc as plsc

VOCAB = 1000000
EMBED_DIM = 64
BATCH = 4096
P_LEN = 50

NUM_IDX = BATCH * P_LEN
NUM_WORKERS = 32
PER_WORKER = NUM_IDX // NUM_WORKERS
CHUNK = 128
NCHUNK = PER_WORKER // CHUNK
NBUF = 5
LOOKAHEAD = 2


def _build_gather():
    mesh = plsc.VectorSubcoreMesh(core_axis_name="c", subcore_axis_name="s")

    scratch = [pltpu.VMEM((NCHUNK, CHUNK), jnp.int32)]
    scratch += [pltpu.VMEM((CHUNK, EMBED_DIM), jnp.float32) for _ in range(NBUF)]
    scratch += [pltpu.SemaphoreType.DMA for _ in range(2 * NBUF)]

    @functools.partial(
        pl.kernel,
        mesh=mesh,
        compiler_params=pltpu.CompilerParams(use_tc_tiling_on_sc=False),
        out_type=jax.ShapeDtypeStruct((NUM_IDX, EMBED_DIM), jnp.float32),
        scratch_types=scratch,
    )
    def gather_kernel(table_hbm, idx_hbm, out_hbm, idx_v, *bufs_and_sems):
        rows = bufs_and_sems[:NBUF]
        sem_g = bufs_and_sems[NBUF:2 * NBUF]
        sem_w = bufs_and_sems[2 * NBUF:]

        wid = lax.axis_index("s") * 2 + lax.axis_index("c")
        base = wid * PER_WORKER
        pltpu.sync_copy(idx_hbm.at[wid], idx_v)

        def fire_gather(f, bf):
            pltpu.async_copy(table_hbm.at[idx_v.at[f]], rows[bf], sem_g[bf])

        def wait_gather(bf):
            pltpu.make_async_copy(
                table_hbm.at[idx_v.at[0]], rows[bf], sem_g[bf]
            ).wait()

        def out_slice(g):
            return out_hbm.at[pl.ds(base + g * CHUNK, CHUNK)]

        def fire_write(g, b):
            pltpu.async_copy(rows[b], out_slice(g), sem_w[b])

        def wait_write(b):
            pltpu.make_async_copy(rows[b], out_slice(0), sem_w[b]).wait()

        for b in range(LOOKAHEAD):
            fire_gather(b, b)

        def body(t, carry):
            for b in range(NBUF):
                g = t * NBUF + b
                f = g + LOOKAHEAD
                bf = (b + LOOKAHEAD) % NBUF

                @pl.when(f < NCHUNK)
                def _():
                    @pl.when(f >= NBUF)
                    def _():
                        wait_write(bf)

                    fire_gather(f, bf)

                wait_gather(b)
                fire_write(g, b)
            return carry

        lax.fori_loop(0, NCHUNK // NBUF, body, 0)

        for b in range(NBUF):
            wait_write(b)

    return gather_kernel


_gather = _build_gather()


def kernel(x, pw_idxs, qw_idxs, p_mask, q_mask, word_vectors):
    idx = x.astype(jnp.int32).reshape(NUM_WORKERS, NCHUNK, CHUNK)
    out = _gather(word_vectors, idx)
    return out.reshape(BATCH, P_LEN, EMBED_DIM)
